# baseline (speedup 1.0000x reference)
import functools

import jax
import jax.numpy as jnp
from jax import lax
from jax.experimental import pallas as pl
from jax.experimental.pallas import tpu as pltpu
from jax.experimental.pallas import tpu_sc as plsc

B = 4096
E = 32
L_TOK = 50
KN = 64
WS = (3, 4, 5)
BB = 256
NBLK = B // BB
CH = 128


def _sc_gather(ctab, tok_t, mtab, mids):
    info = plsc.get_sparse_core_info()
    nc, ns = info.num_cores, info.num_subcores
    nw = nc * ns
    n_com = tok_t.shape[0]
    com_pw = n_com // nw
    n_ch = com_pw // CH
    id_pw = B // nw

    mesh = plsc.VectorSubcoreMesh(core_axis_name="c", subcore_axis_name="s")

    @functools.partial(
        pl.kernel,
        out_type=(
            jax.ShapeDtypeStruct((n_com, E), jnp.float32),
            jax.ShapeDtypeStruct((B, E), jnp.float32),
        ),
        mesh=mesh,
        scratch_types=[
            pltpu.VMEM((com_pw,), jnp.int32),
            pltpu.VMEM((id_pw,), jnp.int32),
            pltpu.VMEM((CH, E), jnp.float32),
            pltpu.VMEM((CH, E), jnp.float32),
            pltpu.VMEM((id_pw, E), jnp.float32),
            pltpu.SemaphoreType.DMA,
            pltpu.SemaphoreType.DMA,
        ],
        compiler_params=pltpu.CompilerParams(use_tc_tiling_on_sc=False),
    )
    def k(ctab_h, tok_h, mtab_h, mids_h, out_c, out_m,
          idx_v, idx_s, buf0, buf1, rows_s, sem0, sem1):
        wid = lax.axis_index("s") * nc + lax.axis_index("c")
        cbase = wid * com_pw
        pltpu.sync_copy(tok_h.at[pl.ds(cbase, com_pw)], idx_v)

        bufs, sems = (buf0, buf1), (sem0, sem1)

        def start(c):
            off = c * CH
            return pltpu.async_copy(
                ctab_h.at[idx_v.at[pl.ds(off, CH)]], bufs[c % 2], sems[c % 2])

        desc = start(0)
        for c in range(n_ch):
            nxt = start(c + 1) if c + 1 < n_ch else None
            desc.wait()
            pltpu.sync_copy(bufs[c % 2], out_c.at[pl.ds(cbase + c * CH, CH)])
            desc = nxt

        ibase = wid * id_pw
        pltpu.sync_copy(mids_h.at[pl.ds(ibase, id_pw)], idx_s)
        pltpu.async_copy(mtab_h.at[idx_s], rows_s, sem0).wait()
        pltpu.sync_copy(rows_s, out_m.at[pl.ds(ibase, id_pw)])

    return k(ctab, tok_t, mtab, mids)


def _sc_uid(utab, uids):
    info = plsc.get_sparse_core_info()
    nc, ns = info.num_cores, info.num_subcores
    nw = nc * ns
    id_pw = B // nw

    mesh = plsc.VectorSubcoreMesh(core_axis_name="c", subcore_axis_name="s")

    @functools.partial(
        pl.kernel,
        out_type=jax.ShapeDtypeStruct((B, E), jnp.float32),
        mesh=mesh,
        scratch_types=[
            pltpu.VMEM((id_pw,), jnp.int32),
            pltpu.VMEM((id_pw, E), jnp.float32),
            pltpu.SemaphoreType.DMA,
        ],
        compiler_params=pltpu.CompilerParams(use_tc_tiling_on_sc=False),
    )
    def k(utab_h, uids_h, out_u, idx_s, rows_s, sem):
        wid = lax.axis_index("s") * nc + lax.axis_index("c")
        ibase = wid * id_pw
        pltpu.sync_copy(uids_h.at[pl.ds(ibase, id_pw)], idx_s)
        pltpu.async_copy(utab_h.at[idx_s], rows_s, sem).wait()
        pltpu.sync_copy(rows_s, out_u.at[pl.ds(ibase, id_pw)])

    return k(utab, uids)


LU = 48


def _tc_movie_body(xt_ref, mid_ref, idxt_ref, mtt_ref,
                   wall, ball, lim, fcw, fcb,
                   mfc1w, mfc1b, mfc2w, mfc2b, mfc3w, mfc3b, mf_out):
    x = xt_ref[...]
    xp = jnp.concatenate(
        [x, jnp.zeros((LU + 5 - L_TOK - 1, BB, E), jnp.float32)], axis=0)
    u = jnp.concatenate([xp[j:j + LU] for j in range(5)], axis=2)
    y = jnp.dot(u.reshape(LU * BB, 5 * E), wall[...],
                preferred_element_type=jnp.float32) + ball[...]
    y = jnp.maximum(y, 0.0).reshape(LU, BB, 3 * KN)
    iota_l = lax.broadcasted_iota(jnp.int32, (LU, 1, 3 * KN), 0)
    y = jnp.where(iota_l < lim[...][None, :, :], y, 0.0)
    feat = jnp.max(y, axis=0)
    mc = jnp.dot(feat, fcw[...], preferred_element_type=jnp.float32) + fcb[...]

    idxt = idxt_ref[0]
    cnt_t = None
    iota34 = lax.broadcasted_iota(jnp.int32, (34, BB), 0)
    for j in range(8):
        oh = (iota34 == idxt[j:j + 1]).astype(jnp.float32)
        cnt_t = oh if cnt_t is None else cnt_t + oh
    mt_e = lax.dot_general(cnt_t, mtt_ref[...], (((0,), (0,)), ((), ())),
                           preferred_element_type=jnp.float32)

    mid_e = mid_ref[...]
    mf = jnp.dot(jnp.concatenate([mt_e, mid_e], axis=1), mfc1w[...],
                 preferred_element_type=jnp.float32) + mfc1b[...]
    mf = jnp.dot(jnp.concatenate([mf, mc], axis=1), mfc2w[...],
                 preferred_element_type=jnp.float32) + mfc2b[...]
    mf = jnp.dot(mf, mfc3w[...], preferred_element_type=jnp.float32) + mfc3b[...]
    mf_out[...] = mf


FIRE = 16


def _tc_user_body(uids_ref, idxt_ref, stt_ref, ufc1w, ufc1b, ufc2w, ufc2b,
                  mf_ref, tab_ref, out_ref, blk, sems):
    cols = []
    for g in range(BB // FIRE):
        descs = []
        for j in range(FIRE):
            v = uids_ref[0, 0, g * FIRE + j]
            s = pl.multiple_of((v // 128) * 128, 128)
            d = pltpu.make_async_copy(tab_ref.at[:, pl.ds(s, 128)],
                                      blk.at[j], sems.at[j])
            d.start()
            descs.append(d)
        for j in range(FIRE):
            descs[j].wait()
            v = uids_ref[0, 0, g * FIRE + j]
            mask = (lax.broadcasted_iota(jnp.int32, (1, 128), 1)
                    == v % 128).astype(jnp.float32)
            cols.append(jnp.sum(blk[j] * mask, axis=1, keepdims=True))
    uid_t = jnp.concatenate(cols, axis=1)

    idxt = idxt_ref[0]
    iota11 = lax.broadcasted_iota(jnp.int32, (11, BB), 0)
    oh_s = (iota11 == idxt[8:9]).astype(jnp.float32)
    ust_t = lax.dot_general(stt_ref[...], oh_s, (((0,), (0,)), ((), ())),
                            preferred_element_type=jnp.float32)
    uf = lax.dot_general(jnp.concatenate([uid_t, ust_t], axis=0), ufc1w[...],
                         (((0,), (0,)), ((), ())),
                         preferred_element_type=jnp.float32) + ufc1b[...]
    uf = jnp.dot(uf, ufc2w[...], preferred_element_type=jnp.float32) + ufc2b[...]
    ret = jnp.sum(mf_ref[...] * uf, axis=1)
    out_ref[0] = (jax.nn.sigmoid(ret) * 5.0).reshape(1, BB)


def _full(shape):
    return pl.BlockSpec(shape, lambda i, _s=shape: tuple(0 for _ in _s))


def _tc_movie(xt, mid_e, idxt, mtt, args):
    in_specs = [
        pl.BlockSpec((L_TOK, BB, E), lambda i: (0, i, 0)),
        pl.BlockSpec((BB, E), lambda i: (i, 0)),
        pl.BlockSpec((1, 16, BB), lambda i: (i, 0, 0)),
        _full((34, E)),
        _full((5 * E, 3 * KN)),
        _full((1, 3 * KN)),
        _full((1, 3 * KN)),
        _full((3 * KN, 32)), _full((1, 32)),
        _full((2 * E, 32)), _full((1, 32)),
        _full((64, 32)), _full((1, 32)),
        _full((32, 16)), _full((1, 16)),
    ]
    return pl.pallas_call(
        _tc_movie_body,
        grid=(NBLK,),
        in_specs=in_specs,
        out_specs=pl.BlockSpec((BB, 16), lambda i: (i, 0)),
        out_shape=jax.ShapeDtypeStruct((B, 16), jnp.float32),
    )(xt, mid_e, idxt, mtt, *args)


def _tc_user(uids3, idxt, stt, ufc1w, ufc1b, ufc2w, ufc2b, mf, tab_t):
    in_specs = [
        pl.BlockSpec((1, 1, BB), lambda i: (i, 0, 0),
                     memory_space=pltpu.SMEM),
        pl.BlockSpec((1, 16, BB), lambda i: (i, 0, 0)),
        _full((11, E)),
        _full((2 * E, 32)), _full((1, 32)),
        _full((32, 16)), _full((1, 16)),
        pl.BlockSpec((BB, 16), lambda i: (i, 0)),
        pl.BlockSpec(memory_space=pltpu.MemorySpace.HBM),
    ]
    return pl.pallas_call(
        _tc_user_body,
        grid=(NBLK,),
        in_specs=in_specs,
        out_specs=pl.BlockSpec((1, 1, BB), lambda i: (i, 0, 0)),
        out_shape=jax.ShapeDtypeStruct((NBLK, 1, BB), jnp.float32),
        scratch_shapes=[pltpu.VMEM((FIRE, E, 128), jnp.float32),
                        pltpu.SemaphoreType.DMA((FIRE,))],
    )(uids3, idxt, stt, ufc1w, ufc1b, ufc2w, ufc2b, mf, tab_t)


def kernel(user_ids, user_socialtype, movie_ids, movie_types, movie_comments,
           socialtype_table, uid_table, movie_types_table, movie_id_table,
           comments_table, conv_w0, conv_b0, conv_w1, conv_b1, conv_w2, conv_b2,
           cnn_fc_w, cnn_fc_b, user_fc1_w, user_fc1_b, user_fc2_w, user_fc2_b,
           movie_fc1_w, movie_fc1_b, movie_fc2_w, movie_fc2_b,
           movie_fc3_w, movie_fc3_b):
    i32 = jnp.int32
    tok_t = movie_comments.astype(i32).T.reshape(-1)
    com_rows, mid_e = _sc_gather(comments_table, tok_t,
                                 movie_id_table, movie_ids.astype(i32))
    xt = com_rows.reshape(L_TOK, B, E)

    idxt = jnp.concatenate([
        movie_types.astype(i32).T,
        user_socialtype.astype(i32)[None, :],
        jnp.zeros((7, B), i32),
    ], axis=0).reshape(16, NBLK, BB).transpose(1, 0, 2)

    wjs = []
    for wsz, cw in ((3, conv_w0), (4, conv_w1), (5, conv_w2)):
        wj = jnp.transpose(cw[:, 0], (1, 2, 0)).reshape(wsz * E, KN)
        wjs.append(jnp.pad(wj, ((0, 5 * E - wsz * E), (0, 0))))
    wall = jnp.concatenate(wjs, axis=1)
    ball = jnp.concatenate([conv_b0, conv_b1, conv_b2]).reshape(1, 3 * KN)
    lim = jnp.concatenate([
        jnp.full((KN,), LU, i32), jnp.full((KN,), LU - 1, i32),
        jnp.full((KN,), LU - 2, i32)]).reshape(1, 3 * KN)

    margs = [wall, ball, lim,
             cnn_fc_w, cnn_fc_b.reshape(1, -1),
             movie_fc1_w, movie_fc1_b.reshape(1, -1),
             movie_fc2_w, movie_fc2_b.reshape(1, -1),
             movie_fc3_w, movie_fc3_b.reshape(1, -1)]
    mf = _tc_movie(xt, mid_e, idxt, movie_types_table, margs)
    uids3 = user_ids.astype(i32).reshape(NBLK, 1, BB)
    out = _tc_user(uids3, idxt, socialtype_table,
                   user_fc1_w, user_fc1_b.reshape(1, -1),
                   user_fc2_w, user_fc2_b.reshape(1, -1), mf, uid_table.T)
    return out.reshape(B)


# speedup vs baseline: 1.1242x; 1.1242x over previous
import functools

import jax
import jax.numpy as jnp
from jax import lax
from jax.experimental import pallas as pl
from jax.experimental.pallas import tpu as pltpu
from jax.experimental.pallas import tpu_sc as plsc

B = 4096
E = 32
L_TOK = 50
KN = 64
WS = (3, 4, 5)
BB = 256
NBLK = B // BB
CH = 128


def _sc_gather(ctab, tok_t, mtab, mids):
    info = plsc.get_sparse_core_info()
    nc, ns = info.num_cores, info.num_subcores
    nw = nc * ns
    n_com = tok_t.shape[0]
    com_pw = n_com // nw
    n_ch = com_pw // CH
    id_pw = B // nw

    mesh = plsc.VectorSubcoreMesh(core_axis_name="c", subcore_axis_name="s")

    @functools.partial(
        pl.kernel,
        out_type=(
            jax.ShapeDtypeStruct((n_com, E), jnp.float32),
            jax.ShapeDtypeStruct((B, E), jnp.float32),
        ),
        mesh=mesh,
        scratch_types=[
            pltpu.VMEM((com_pw,), jnp.int32),
            pltpu.VMEM((id_pw,), jnp.int32),
            pltpu.VMEM((CH, E), jnp.float32),
            pltpu.VMEM((CH, E), jnp.float32),
            pltpu.VMEM((id_pw, E), jnp.float32),
            pltpu.SemaphoreType.DMA,
            pltpu.SemaphoreType.DMA,
        ],
        compiler_params=pltpu.CompilerParams(use_tc_tiling_on_sc=False),
    )
    def k(ctab_h, tok_h, mtab_h, mids_h, out_c, out_m,
          idx_v, idx_s, buf0, buf1, rows_s, sem0, sem1):
        wid = lax.axis_index("s") * nc + lax.axis_index("c")
        cbase = wid * com_pw
        pltpu.sync_copy(tok_h.at[pl.ds(cbase, com_pw)], idx_v)

        bufs, sems = (buf0, buf1), (sem0, sem1)

        def start(c):
            off = c * CH
            return pltpu.async_copy(
                ctab_h.at[idx_v.at[pl.ds(off, CH)]], bufs[c % 2], sems[c % 2])

        desc = start(0)
        for c in range(n_ch):
            nxt = start(c + 1) if c + 1 < n_ch else None
            desc.wait()
            pltpu.sync_copy(bufs[c % 2], out_c.at[pl.ds(cbase + c * CH, CH)])
            desc = nxt

        ibase = wid * id_pw
        pltpu.sync_copy(mids_h.at[pl.ds(ibase, id_pw)], idx_s)
        pltpu.async_copy(mtab_h.at[idx_s], rows_s, sem0).wait()
        pltpu.sync_copy(rows_s, out_m.at[pl.ds(ibase, id_pw)])

    return k(ctab, tok_t, mtab, mids)


def _sc_uid(utab, uids):
    info = plsc.get_sparse_core_info()
    nc, ns = info.num_cores, info.num_subcores
    nw = nc * ns
    id_pw = B // nw

    mesh = plsc.VectorSubcoreMesh(core_axis_name="c", subcore_axis_name="s")

    @functools.partial(
        pl.kernel,
        out_type=jax.ShapeDtypeStruct((B, E), jnp.float32),
        mesh=mesh,
        scratch_types=[
            pltpu.VMEM((id_pw,), jnp.int32),
            pltpu.VMEM((id_pw, E), jnp.float32),
            pltpu.SemaphoreType.DMA,
        ],
        compiler_params=pltpu.CompilerParams(use_tc_tiling_on_sc=False),
    )
    def k(utab_h, uids_h, out_u, idx_s, rows_s, sem):
        wid = lax.axis_index("s") * nc + lax.axis_index("c")
        ibase = wid * id_pw
        pltpu.sync_copy(uids_h.at[pl.ds(ibase, id_pw)], idx_s)
        pltpu.async_copy(utab_h.at[idx_s], rows_s, sem).wait()
        pltpu.sync_copy(rows_s, out_u.at[pl.ds(ibase, id_pw)])

    return k(utab, uids)


LU = 48


def _tc_movie_body(xt_ref, mid_ref, idxt_ref, mtt_ref,
                   wall, ball, lim, fcw, fcb,
                   mfc1w, mfc1b, mfc2w, mfc2b, mfc3w, mfc3b, mf_out):
    x = xt_ref[...]
    xp = jnp.concatenate(
        [x, jnp.zeros((LU + 5 - L_TOK - 1, BB, E), jnp.float32)], axis=0)
    u = jnp.concatenate([xp[j:j + LU] for j in range(5)], axis=2)
    y = jnp.dot(u.reshape(LU * BB, 5 * E), wall[...],
                preferred_element_type=jnp.float32) + ball[...]
    y = jnp.maximum(y, 0.0).reshape(LU, BB, 3 * KN)
    iota_l = lax.broadcasted_iota(jnp.int32, (LU, 1, 3 * KN), 0)
    y = jnp.where(iota_l < lim[...][None, :, :], y, 0.0)
    feat = jnp.max(y, axis=0)
    mc = jnp.dot(feat, fcw[...], preferred_element_type=jnp.float32) + fcb[...]

    idxt = idxt_ref[0]
    cnt_t = None
    iota34 = lax.broadcasted_iota(jnp.int32, (34, BB), 0)
    for j in range(8):
        oh = (iota34 == idxt[j:j + 1]).astype(jnp.float32)
        cnt_t = oh if cnt_t is None else cnt_t + oh
    mt_e = lax.dot_general(cnt_t, mtt_ref[...], (((0,), (0,)), ((), ())),
                           preferred_element_type=jnp.float32)

    mid_e = mid_ref[...]
    mf = jnp.dot(jnp.concatenate([mt_e, mid_e], axis=1), mfc1w[...],
                 preferred_element_type=jnp.float32) + mfc1b[...]
    mf = jnp.dot(jnp.concatenate([mf, mc], axis=1), mfc2w[...],
                 preferred_element_type=jnp.float32) + mfc2b[...]
    mf = jnp.dot(mf, mfc3w[...], preferred_element_type=jnp.float32) + mfc3b[...]
    mf_out[...] = mf


def _tc_user_body(uid_ref, idxt_ref, stt_ref, ufc1w, ufc1b, ufc2w, ufc2b,
                  mf_ref, out_ref):
    idxt = idxt_ref[0]
    iota11 = lax.broadcasted_iota(jnp.int32, (11, BB), 0)
    oh_s = (iota11 == idxt[8:9]).astype(jnp.float32)
    ust_e = lax.dot_general(oh_s, stt_ref[...], (((0,), (0,)), ((), ())),
                            preferred_element_type=jnp.float32)
    uid_e = uid_ref[...]
    uf = jnp.dot(jnp.concatenate([uid_e, ust_e], axis=1), ufc1w[...],
                 preferred_element_type=jnp.float32) + ufc1b[...]
    uf = jnp.dot(uf, ufc2w[...], preferred_element_type=jnp.float32) + ufc2b[...]
    ret = jnp.sum(mf_ref[...] * uf, axis=1)
    out_ref[0] = (jax.nn.sigmoid(ret) * 5.0).reshape(1, BB)


def _full(shape):
    return pl.BlockSpec(shape, lambda i, _s=shape: tuple(0 for _ in _s))


def _tc_movie(xt, mid_e, idxt, mtt, args):
    in_specs = [
        pl.BlockSpec((L_TOK, BB, E), lambda i: (0, i, 0)),
        pl.BlockSpec((BB, E), lambda i: (i, 0)),
        pl.BlockSpec((1, 16, BB), lambda i: (i, 0, 0)),
        _full((34, E)),
        _full((5 * E, 3 * KN)),
        _full((1, 3 * KN)),
        _full((1, 3 * KN)),
        _full((3 * KN, 32)), _full((1, 32)),
        _full((2 * E, 32)), _full((1, 32)),
        _full((64, 32)), _full((1, 32)),
        _full((32, 16)), _full((1, 16)),
    ]
    return pl.pallas_call(
        _tc_movie_body,
        grid=(NBLK,),
        in_specs=in_specs,
        out_specs=pl.BlockSpec((BB, 16), lambda i: (i, 0)),
        out_shape=jax.ShapeDtypeStruct((B, 16), jnp.float32),
    )(xt, mid_e, idxt, mtt, *args)


def _tc_user(uid_e, idxt, stt, ufc1w, ufc1b, ufc2w, ufc2b, mf):
    in_specs = [
        pl.BlockSpec((BB, E), lambda i: (i, 0)),
        pl.BlockSpec((1, 16, BB), lambda i: (i, 0, 0)),
        _full((11, E)),
        _full((2 * E, 32)), _full((1, 32)),
        _full((32, 16)), _full((1, 16)),
        pl.BlockSpec((BB, 16), lambda i: (i, 0)),
    ]
    return pl.pallas_call(
        _tc_user_body,
        grid=(NBLK,),
        in_specs=in_specs,
        out_specs=pl.BlockSpec((1, 1, BB), lambda i: (i, 0, 0)),
        out_shape=jax.ShapeDtypeStruct((NBLK, 1, BB), jnp.float32),
    )(uid_e, idxt, stt, ufc1w, ufc1b, ufc2w, ufc2b, mf)


def kernel(user_ids, user_socialtype, movie_ids, movie_types, movie_comments,
           socialtype_table, uid_table, movie_types_table, movie_id_table,
           comments_table, conv_w0, conv_b0, conv_w1, conv_b1, conv_w2, conv_b2,
           cnn_fc_w, cnn_fc_b, user_fc1_w, user_fc1_b, user_fc2_w, user_fc2_b,
           movie_fc1_w, movie_fc1_b, movie_fc2_w, movie_fc2_b,
           movie_fc3_w, movie_fc3_b):
    i32 = jnp.int32
    tok_t = movie_comments.astype(i32).T.reshape(-1)
    com_rows, mid_e = _sc_gather(comments_table, tok_t,
                                 movie_id_table, movie_ids.astype(i32))
    uid_e = _sc_uid(uid_table, user_ids.astype(i32))
    xt = com_rows.reshape(L_TOK, B, E)

    idxt = jnp.concatenate([
        movie_types.astype(i32).T,
        user_socialtype.astype(i32)[None, :],
        jnp.zeros((7, B), i32),
    ], axis=0).reshape(16, NBLK, BB).transpose(1, 0, 2)

    wjs = []
    for wsz, cw in ((3, conv_w0), (4, conv_w1), (5, conv_w2)):
        wj = jnp.transpose(cw[:, 0], (1, 2, 0)).reshape(wsz * E, KN)
        wjs.append(jnp.pad(wj, ((0, 5 * E - wsz * E), (0, 0))))
    wall = jnp.concatenate(wjs, axis=1)
    ball = jnp.concatenate([conv_b0, conv_b1, conv_b2]).reshape(1, 3 * KN)
    lim = jnp.concatenate([
        jnp.full((KN,), LU, i32), jnp.full((KN,), LU - 1, i32),
        jnp.full((KN,), LU - 2, i32)]).reshape(1, 3 * KN)

    margs = [wall, ball, lim,
             cnn_fc_w, cnn_fc_b.reshape(1, -1),
             movie_fc1_w, movie_fc1_b.reshape(1, -1),
             movie_fc2_w, movie_fc2_b.reshape(1, -1),
             movie_fc3_w, movie_fc3_b.reshape(1, -1)]
    mf = _tc_movie(xt, mid_e, idxt, movie_types_table, margs)
    out = _tc_user(uid_e, idxt, socialtype_table,
                   user_fc1_w, user_fc1_b.reshape(1, -1),
                   user_fc2_w, user_fc2_b.reshape(1, -1), mf)
    return out.reshape(B)


# speedup vs baseline: 1.1820x; 1.0514x over previous
import functools

import jax
import jax.numpy as jnp
from jax import lax
from jax.experimental import pallas as pl
from jax.experimental.pallas import tpu as pltpu
from jax.experimental.pallas import tpu_sc as plsc

B = 4096
E = 32
L_TOK = 50
KN = 64
WS = (3, 4, 5)
BB = 256
NBLK = B // BB
CH = 128


def _sc_gather(ctab, tok_t, mtab, mids):
    info = plsc.get_sparse_core_info()
    nc, ns = info.num_cores, info.num_subcores
    nw = nc * ns
    n_com = tok_t.shape[0]
    com_pw = n_com // nw
    n_ch = com_pw // CH
    id_pw = B // nw

    mesh = plsc.VectorSubcoreMesh(core_axis_name="c", subcore_axis_name="s")

    @functools.partial(
        pl.kernel,
        out_type=(
            jax.ShapeDtypeStruct((n_com, E), jnp.float32),
            jax.ShapeDtypeStruct((B, E), jnp.float32),
        ),
        mesh=mesh,
        scratch_types=[
            pltpu.VMEM((com_pw,), jnp.int32),
            pltpu.VMEM((id_pw,), jnp.int32),
            pltpu.VMEM((CH, E), jnp.float32),
            pltpu.VMEM((CH, E), jnp.float32),
            pltpu.VMEM((id_pw, E), jnp.float32),
            pltpu.SemaphoreType.DMA,
            pltpu.SemaphoreType.DMA,
        ],
        compiler_params=pltpu.CompilerParams(use_tc_tiling_on_sc=False),
    )
    def k(ctab_h, tok_h, mtab_h, mids_h, out_c, out_m,
          idx_v, idx_s, buf0, buf1, rows_s, sem0, sem1):
        wid = lax.axis_index("s") * nc + lax.axis_index("c")
        cbase = wid * com_pw
        pltpu.sync_copy(tok_h.at[pl.ds(cbase, com_pw)], idx_v)

        bufs, sems = (buf0, buf1), (sem0, sem1)

        def start(c):
            off = c * CH
            return pltpu.async_copy(
                ctab_h.at[idx_v.at[pl.ds(off, CH)]], bufs[c % 2], sems[c % 2])

        desc = start(0)
        for c in range(n_ch):
            nxt = start(c + 1) if c + 1 < n_ch else None
            desc.wait()
            pltpu.sync_copy(bufs[c % 2], out_c.at[pl.ds(cbase + c * CH, CH)])
            desc = nxt

        ibase = wid * id_pw
        pltpu.sync_copy(mids_h.at[pl.ds(ibase, id_pw)], idx_s)
        pltpu.async_copy(mtab_h.at[idx_s], rows_s, sem0).wait()
        pltpu.sync_copy(rows_s, out_m.at[pl.ds(ibase, id_pw)])

    return k(ctab, tok_t, mtab, mids)


def _sc_uid(utab, uids):
    info = plsc.get_sparse_core_info()
    nc, ns = info.num_cores, info.num_subcores
    nw = nc * ns
    id_pw = B // nw

    mesh = plsc.VectorSubcoreMesh(core_axis_name="c", subcore_axis_name="s")

    @functools.partial(
        pl.kernel,
        out_type=jax.ShapeDtypeStruct((B, E), jnp.float32),
        mesh=mesh,
        scratch_types=[
            pltpu.VMEM((id_pw,), jnp.int32),
            pltpu.VMEM((id_pw, E), jnp.float32),
            pltpu.SemaphoreType.DMA,
        ],
        compiler_params=pltpu.CompilerParams(use_tc_tiling_on_sc=False),
    )
    def k(utab_h, uids_h, out_u, idx_s, rows_s, sem):
        wid = lax.axis_index("s") * nc + lax.axis_index("c")
        ibase = wid * id_pw
        pltpu.sync_copy(uids_h.at[pl.ds(ibase, id_pw)], idx_s)
        pltpu.async_copy(utab_h.at[idx_s], rows_s, sem).wait()
        pltpu.sync_copy(rows_s, out_u.at[pl.ds(ibase, id_pw)])

    return k(utab, uids)


LU = 48


def _tc_movie_body(xt_ref, mid_ref, idxt_ref, mtt_ref,
                   wall, ball, lim, fcw, fcb,
                   mfc1w, mfc1b, mfc2w, mfc2b, mfc3w, mfc3b, mf_out):
    x = xt_ref[...].astype(jnp.bfloat16)
    xp = jnp.concatenate(
        [x, jnp.zeros((LU + 5 - L_TOK - 1, BB, E), jnp.bfloat16)], axis=0)
    u = jnp.concatenate([xp[j:j + LU] for j in range(5)], axis=2)
    y = jnp.dot(u.reshape(LU * BB, 5 * E), wall[...].astype(jnp.bfloat16),
                preferred_element_type=jnp.float32) + ball[...]
    y = jnp.maximum(y, 0.0).reshape(LU, BB, 3 * KN)
    iota_l = lax.broadcasted_iota(jnp.int32, (LU, 1, 3 * KN), 0)
    y = jnp.where(iota_l < lim[...][None, :, :], y, 0.0)
    feat = jnp.max(y, axis=0)
    mc = jnp.dot(feat, fcw[...], preferred_element_type=jnp.float32) + fcb[...]

    idxt = idxt_ref[0]
    cnt_t = None
    iota34 = lax.broadcasted_iota(jnp.int32, (34, BB), 0)
    for j in range(8):
        oh = (iota34 == idxt[j:j + 1]).astype(jnp.float32)
        cnt_t = oh if cnt_t is None else cnt_t + oh
    mt_e = lax.dot_general(cnt_t, mtt_ref[...], (((0,), (0,)), ((), ())),
                           preferred_element_type=jnp.float32)

    mid_e = mid_ref[...]
    mf = jnp.dot(jnp.concatenate([mt_e, mid_e], axis=1), mfc1w[...],
                 preferred_element_type=jnp.float32) + mfc1b[...]
    mf = jnp.dot(jnp.concatenate([mf, mc], axis=1), mfc2w[...],
                 preferred_element_type=jnp.float32) + mfc2b[...]
    mf = jnp.dot(mf, mfc3w[...], preferred_element_type=jnp.float32) + mfc3b[...]
    mf_out[...] = mf


def _tc_user_body(uid_ref, idxt_ref, stt_ref, ufc1w, ufc1b, ufc2w, ufc2b,
                  mf_ref, out_ref):
    idxt = idxt_ref[0]
    iota11 = lax.broadcasted_iota(jnp.int32, (11, BB), 0)
    oh_s = (iota11 == idxt[8:9]).astype(jnp.float32)
    ust_e = lax.dot_general(oh_s, stt_ref[...], (((0,), (0,)), ((), ())),
                            preferred_element_type=jnp.float32)
    uid_e = uid_ref[...]
    uf = jnp.dot(jnp.concatenate([uid_e, ust_e], axis=1), ufc1w[...],
                 preferred_element_type=jnp.float32) + ufc1b[...]
    uf = jnp.dot(uf, ufc2w[...], preferred_element_type=jnp.float32) + ufc2b[...]
    ret = jnp.sum(mf_ref[...] * uf, axis=1)
    out_ref[0] = (jax.nn.sigmoid(ret) * 5.0).reshape(1, BB)


def _full(shape):
    return pl.BlockSpec(shape, lambda i, _s=shape: tuple(0 for _ in _s))


def _tc_movie(xt, mid_e, idxt, mtt, args):
    in_specs = [
        pl.BlockSpec((L_TOK, BB, E), lambda i: (0, i, 0)),
        pl.BlockSpec((BB, E), lambda i: (i, 0)),
        pl.BlockSpec((1, 16, BB), lambda i: (i, 0, 0)),
        _full((34, E)),
        _full((5 * E, 3 * KN)),
        _full((1, 3 * KN)),
        _full((1, 3 * KN)),
        _full((3 * KN, 32)), _full((1, 32)),
        _full((2 * E, 32)), _full((1, 32)),
        _full((64, 32)), _full((1, 32)),
        _full((32, 16)), _full((1, 16)),
    ]
    return pl.pallas_call(
        _tc_movie_body,
        grid=(NBLK,),
        in_specs=in_specs,
        out_specs=pl.BlockSpec((BB, 16), lambda i: (i, 0)),
        out_shape=jax.ShapeDtypeStruct((B, 16), jnp.float32),
    )(xt, mid_e, idxt, mtt, *args)


def _tc_user(uid_e, idxt, stt, ufc1w, ufc1b, ufc2w, ufc2b, mf):
    in_specs = [
        pl.BlockSpec((BB, E), lambda i: (i, 0)),
        pl.BlockSpec((1, 16, BB), lambda i: (i, 0, 0)),
        _full((11, E)),
        _full((2 * E, 32)), _full((1, 32)),
        _full((32, 16)), _full((1, 16)),
        pl.BlockSpec((BB, 16), lambda i: (i, 0)),
    ]
    return pl.pallas_call(
        _tc_user_body,
        grid=(NBLK,),
        in_specs=in_specs,
        out_specs=pl.BlockSpec((1, 1, BB), lambda i: (i, 0, 0)),
        out_shape=jax.ShapeDtypeStruct((NBLK, 1, BB), jnp.float32),
    )(uid_e, idxt, stt, ufc1w, ufc1b, ufc2w, ufc2b, mf)


def kernel(user_ids, user_socialtype, movie_ids, movie_types, movie_comments,
           socialtype_table, uid_table, movie_types_table, movie_id_table,
           comments_table, conv_w0, conv_b0, conv_w1, conv_b1, conv_w2, conv_b2,
           cnn_fc_w, cnn_fc_b, user_fc1_w, user_fc1_b, user_fc2_w, user_fc2_b,
           movie_fc1_w, movie_fc1_b, movie_fc2_w, movie_fc2_b,
           movie_fc3_w, movie_fc3_b):
    i32 = jnp.int32
    tok_t = movie_comments.astype(i32).T.reshape(-1)
    com_rows, mid_e = _sc_gather(comments_table, tok_t,
                                 movie_id_table, movie_ids.astype(i32))
    uid_e = _sc_uid(uid_table, user_ids.astype(i32))
    xt = com_rows.reshape(L_TOK, B, E)

    idxt = jnp.concatenate([
        movie_types.astype(i32).T,
        user_socialtype.astype(i32)[None, :],
        jnp.zeros((7, B), i32),
    ], axis=0).reshape(16, NBLK, BB).transpose(1, 0, 2)

    wjs = []
    for wsz, cw in ((3, conv_w0), (4, conv_w1), (5, conv_w2)):
        wj = jnp.transpose(cw[:, 0], (1, 2, 0)).reshape(wsz * E, KN)
        wjs.append(jnp.pad(wj, ((0, 5 * E - wsz * E), (0, 0))))
    wall = jnp.concatenate(wjs, axis=1)
    ball = jnp.concatenate([conv_b0, conv_b1, conv_b2]).reshape(1, 3 * KN)
    lim = jnp.concatenate([
        jnp.full((KN,), LU, i32), jnp.full((KN,), LU - 1, i32),
        jnp.full((KN,), LU - 2, i32)]).reshape(1, 3 * KN)

    margs = [wall, ball, lim,
             cnn_fc_w, cnn_fc_b.reshape(1, -1),
             movie_fc1_w, movie_fc1_b.reshape(1, -1),
             movie_fc2_w, movie_fc2_b.reshape(1, -1),
             movie_fc3_w, movie_fc3_b.reshape(1, -1)]
    mf = _tc_movie(xt, mid_e, idxt, movie_types_table, margs)
    out = _tc_user(uid_e, idxt, socialtype_table,
                   user_fc1_w, user_fc1_b.reshape(1, -1),
                   user_fc2_w, user_fc2_b.reshape(1, -1), mf)
    return out.reshape(B)


# speedup vs baseline: 1.1894x; 1.0063x over previous
import functools

import jax
import jax.numpy as jnp
from jax import lax
from jax.experimental import pallas as pl
from jax.experimental.pallas import tpu as pltpu
from jax.experimental.pallas import tpu_sc as plsc

B = 4096
E = 32
L_TOK = 50
KN = 64
WS = (3, 4, 5)
BB = 512
NBLK = B // BB
CH = 128


def _sc_gather(ctab, tok_t, mtab, mids):
    info = plsc.get_sparse_core_info()
    nc, ns = info.num_cores, info.num_subcores
    nw = nc * ns
    n_com = tok_t.shape[0]
    com_pw = n_com // nw
    n_ch = com_pw // CH
    id_pw = B // nw

    mesh = plsc.VectorSubcoreMesh(core_axis_name="c", subcore_axis_name="s")

    @functools.partial(
        pl.kernel,
        out_type=(
            jax.ShapeDtypeStruct((n_com, E), jnp.float32),
            jax.ShapeDtypeStruct((B, E), jnp.float32),
        ),
        mesh=mesh,
        scratch_types=[
            pltpu.VMEM((com_pw,), jnp.int32),
            pltpu.VMEM((id_pw,), jnp.int32),
            pltpu.VMEM((CH, E), jnp.float32),
            pltpu.VMEM((CH, E), jnp.float32),
            pltpu.VMEM((id_pw, E), jnp.float32),
            pltpu.SemaphoreType.DMA,
            pltpu.SemaphoreType.DMA,
        ],
        compiler_params=pltpu.CompilerParams(use_tc_tiling_on_sc=False),
    )
    def k(ctab_h, tok_h, mtab_h, mids_h, out_c, out_m,
          idx_v, idx_s, buf0, buf1, rows_s, sem0, sem1):
        wid = lax.axis_index("s") * nc + lax.axis_index("c")
        cbase = wid * com_pw
        pltpu.sync_copy(tok_h.at[pl.ds(cbase, com_pw)], idx_v)

        bufs, sems = (buf0, buf1), (sem0, sem1)

        def start(c):
            off = c * CH
            return pltpu.async_copy(
                ctab_h.at[idx_v.at[pl.ds(off, CH)]], bufs[c % 2], sems[c % 2])

        desc = start(0)
        for c in range(n_ch):
            nxt = start(c + 1) if c + 1 < n_ch else None
            desc.wait()
            pltpu.sync_copy(bufs[c % 2], out_c.at[pl.ds(cbase + c * CH, CH)])
            desc = nxt

        ibase = wid * id_pw
        pltpu.sync_copy(mids_h.at[pl.ds(ibase, id_pw)], idx_s)
        pltpu.async_copy(mtab_h.at[idx_s], rows_s, sem0).wait()
        pltpu.sync_copy(rows_s, out_m.at[pl.ds(ibase, id_pw)])

    return k(ctab, tok_t, mtab, mids)


def _sc_uid(utab, uids):
    info = plsc.get_sparse_core_info()
    nc, ns = info.num_cores, info.num_subcores
    nw = nc * ns
    id_pw = B // nw

    mesh = plsc.VectorSubcoreMesh(core_axis_name="c", subcore_axis_name="s")

    @functools.partial(
        pl.kernel,
        out_type=jax.ShapeDtypeStruct((B, E), jnp.float32),
        mesh=mesh,
        scratch_types=[
            pltpu.VMEM((id_pw,), jnp.int32),
            pltpu.VMEM((id_pw, E), jnp.float32),
            pltpu.SemaphoreType.DMA,
        ],
        compiler_params=pltpu.CompilerParams(use_tc_tiling_on_sc=False),
    )
    def k(utab_h, uids_h, out_u, idx_s, rows_s, sem):
        wid = lax.axis_index("s") * nc + lax.axis_index("c")
        ibase = wid * id_pw
        pltpu.sync_copy(uids_h.at[pl.ds(ibase, id_pw)], idx_s)
        pltpu.async_copy(utab_h.at[idx_s], rows_s, sem).wait()
        pltpu.sync_copy(rows_s, out_u.at[pl.ds(ibase, id_pw)])

    return k(utab, uids)


LU = 48


def _tc_movie_body(xt_ref, mid_ref, idxt_ref, mtt_ref,
                   wall, ball, lim, fcw, fcb,
                   mfc1w, mfc1b, mfc2w, mfc2b, mfc3w, mfc3b, mf_out):
    x = xt_ref[...].astype(jnp.bfloat16)
    xp = jnp.concatenate(
        [x, jnp.zeros((LU + 5 - L_TOK - 1, BB, E), jnp.bfloat16)], axis=0)
    u = jnp.concatenate([xp[j:j + LU] for j in range(5)], axis=2)
    y = jnp.dot(u.reshape(LU * BB, 5 * E), wall[...].astype(jnp.bfloat16),
                preferred_element_type=jnp.float32) + ball[...]
    y = jnp.maximum(y, 0.0).reshape(LU, BB, 3 * KN)
    iota_l = lax.broadcasted_iota(jnp.int32, (LU, 1, 3 * KN), 0)
    y = jnp.where(iota_l < lim[...][None, :, :], y, 0.0)
    feat = jnp.max(y, axis=0)
    mc = jnp.dot(feat, fcw[...], preferred_element_type=jnp.float32) + fcb[...]

    idxt = idxt_ref[0]
    cnt_t = None
    iota34 = lax.broadcasted_iota(jnp.int32, (34, BB), 0)
    for j in range(8):
        oh = (iota34 == idxt[j:j + 1]).astype(jnp.float32)
        cnt_t = oh if cnt_t is None else cnt_t + oh
    mt_e = lax.dot_general(cnt_t, mtt_ref[...], (((0,), (0,)), ((), ())),
                           preferred_element_type=jnp.float32)

    mid_e = mid_ref[...]
    mf = jnp.dot(jnp.concatenate([mt_e, mid_e], axis=1), mfc1w[...],
                 preferred_element_type=jnp.float32) + mfc1b[...]
    mf = jnp.dot(jnp.concatenate([mf, mc], axis=1), mfc2w[...],
                 preferred_element_type=jnp.float32) + mfc2b[...]
    mf = jnp.dot(mf, mfc3w[...], preferred_element_type=jnp.float32) + mfc3b[...]
    mf_out[...] = mf


def _tc_user_body(uid_ref, idxt_ref, stt_ref, ufc1w, ufc1b, ufc2w, ufc2b,
                  mf_ref, out_ref):
    idxt = idxt_ref[0]
    iota11 = lax.broadcasted_iota(jnp.int32, (11, BB), 0)
    oh_s = (iota11 == idxt[8:9]).astype(jnp.float32)
    ust_e = lax.dot_general(oh_s, stt_ref[...], (((0,), (0,)), ((), ())),
                            preferred_element_type=jnp.float32)
    uid_e = uid_ref[...]
    uf = jnp.dot(jnp.concatenate([uid_e, ust_e], axis=1), ufc1w[...],
                 preferred_element_type=jnp.float32) + ufc1b[...]
    uf = jnp.dot(uf, ufc2w[...], preferred_element_type=jnp.float32) + ufc2b[...]
    ret = jnp.sum(mf_ref[...] * uf, axis=1)
    out_ref[0] = (jax.nn.sigmoid(ret) * 5.0).reshape(1, BB)


def _full(shape):
    return pl.BlockSpec(shape, lambda i, _s=shape: tuple(0 for _ in _s))


def _tc_movie(xt, mid_e, idxt, mtt, args):
    in_specs = [
        pl.BlockSpec((L_TOK, BB, E), lambda i: (0, i, 0)),
        pl.BlockSpec((BB, E), lambda i: (i, 0)),
        pl.BlockSpec((1, 16, BB), lambda i: (i, 0, 0)),
        _full((34, E)),
        _full((5 * E, 3 * KN)),
        _full((1, 3 * KN)),
        _full((1, 3 * KN)),
        _full((3 * KN, 32)), _full((1, 32)),
        _full((2 * E, 32)), _full((1, 32)),
        _full((64, 32)), _full((1, 32)),
        _full((32, 16)), _full((1, 16)),
    ]
    return pl.pallas_call(
        _tc_movie_body,
        grid=(NBLK,),
        in_specs=in_specs,
        out_specs=pl.BlockSpec((BB, 16), lambda i: (i, 0)),
        out_shape=jax.ShapeDtypeStruct((B, 16), jnp.float32),
    )(xt, mid_e, idxt, mtt, *args)


def _tc_user(uid_e, idxt, stt, ufc1w, ufc1b, ufc2w, ufc2b, mf):
    in_specs = [
        pl.BlockSpec((BB, E), lambda i: (i, 0)),
        pl.BlockSpec((1, 16, BB), lambda i: (i, 0, 0)),
        _full((11, E)),
        _full((2 * E, 32)), _full((1, 32)),
        _full((32, 16)), _full((1, 16)),
        pl.BlockSpec((BB, 16), lambda i: (i, 0)),
    ]
    return pl.pallas_call(
        _tc_user_body,
        grid=(NBLK,),
        in_specs=in_specs,
        out_specs=pl.BlockSpec((1, 1, BB), lambda i: (i, 0, 0)),
        out_shape=jax.ShapeDtypeStruct((NBLK, 1, BB), jnp.float32),
    )(uid_e, idxt, stt, ufc1w, ufc1b, ufc2w, ufc2b, mf)


def kernel(user_ids, user_socialtype, movie_ids, movie_types, movie_comments,
           socialtype_table, uid_table, movie_types_table, movie_id_table,
           comments_table, conv_w0, conv_b0, conv_w1, conv_b1, conv_w2, conv_b2,
           cnn_fc_w, cnn_fc_b, user_fc1_w, user_fc1_b, user_fc2_w, user_fc2_b,
           movie_fc1_w, movie_fc1_b, movie_fc2_w, movie_fc2_b,
           movie_fc3_w, movie_fc3_b):
    i32 = jnp.int32
    tok_t = movie_comments.astype(i32).T.reshape(-1)
    com_rows, mid_e = _sc_gather(comments_table, tok_t,
                                 movie_id_table, movie_ids.astype(i32))
    uid_e = _sc_uid(uid_table, user_ids.astype(i32))
    xt = com_rows.reshape(L_TOK, B, E)

    idxt = jnp.concatenate([
        movie_types.astype(i32).T,
        user_socialtype.astype(i32)[None, :],
        jnp.zeros((7, B), i32),
    ], axis=0).reshape(16, NBLK, BB).transpose(1, 0, 2)

    wjs = []
    for wsz, cw in ((3, conv_w0), (4, conv_w1), (5, conv_w2)):
        wj = jnp.transpose(cw[:, 0], (1, 2, 0)).reshape(wsz * E, KN)
        wjs.append(jnp.pad(wj, ((0, 5 * E - wsz * E), (0, 0))))
    wall = jnp.concatenate(wjs, axis=1)
    ball = jnp.concatenate([conv_b0, conv_b1, conv_b2]).reshape(1, 3 * KN)
    lim = jnp.concatenate([
        jnp.full((KN,), LU, i32), jnp.full((KN,), LU - 1, i32),
        jnp.full((KN,), LU - 2, i32)]).reshape(1, 3 * KN)

    margs = [wall, ball, lim,
             cnn_fc_w, cnn_fc_b.reshape(1, -1),
             movie_fc1_w, movie_fc1_b.reshape(1, -1),
             movie_fc2_w, movie_fc2_b.reshape(1, -1),
             movie_fc3_w, movie_fc3_b.reshape(1, -1)]
    mf = _tc_movie(xt, mid_e, idxt, movie_types_table, margs)
    out = _tc_user(uid_e, idxt, socialtype_table,
                   user_fc1_w, user_fc1_b.reshape(1, -1),
                   user_fc2_w, user_fc2_b.reshape(1, -1), mf)
    return out.reshape(B)
